# Initial kernel scaffold; baseline (speedup 1.0000x reference)
import jax
import jax.numpy as jnp
from jax.experimental import pallas as pl


def kernel(template_id_int, atom_role, tables):
    raise NotImplementedError("write your pallas kernel here")


# speedup vs baseline: 2.0453x; 2.0453x over previous
import functools

import jax
import jax.numpy as jnp
from jax import lax
from jax.experimental import pallas as pl
from jax.experimental.pallas import tpu as pltpu
from jax.experimental.pallas import tpu_sc as plsc

_NUM_TEMPLATES = 1000
_NUM_ROLES = 64
_ROLE_DIM = 128
_B = 4096
_N = 50

_NW = 32
_RPW = _B // _NW
_CHUNK = _RPW * _N
_G = 128
_NG = _CHUNK // _G


def _sc_body(tid_hbm, role_hbm, tbl_hbm, out_hbm,
             tid_v, role_v, idx_v, buf0, buf1, sem0, sem1):
    cid = lax.axis_index("c")
    sid = lax.axis_index("s")
    wid = cid * 16 + sid
    base = wid * _CHUNK
    rbase = wid * _RPW

    pltpu.sync_copy(tid_hbm.at[pl.ds(rbase, _RPW)], tid_v)
    pltpu.sync_copy(role_hbm.at[pl.ds(base, _CHUNK)], role_v)

    lanes = lax.iota(jnp.int32, 16)

    def idx_body(k, _):
        p0 = k * 16
        pv = p0 + lanes
        iv = lax.shift_right_logical(pv * 41944, 21)
        tv = plsc.load_gather(tid_v, [iv])
        rv = role_v[pl.ds(p0, 16)]
        idx_v[k // (_G // 16), pl.ds((k % (_G // 16)) * 16, 16)] = (
            tv * _NUM_ROLES + rv)
        return 0

    lax.fori_loop(0, _CHUNK // 16, idx_body, 0)

    bufs = (buf0, buf1)
    sems = (sem0, sem1)

    def fire(g, b):
        pltpu.async_copy(tbl_hbm.at[idx_v.at[g]], bufs[b], sems[b])

    fire(0, 0)

    def g_body(i, _):
        for b in range(2):
            gg = i * 2 + b

            @pl.when(gg + 1 < _NG)
            def _():
                fire(gg + 1, 1 - b)

            pltpu.make_async_copy(
                tbl_hbm.at[idx_v.at[gg]], bufs[b], sems[b]).wait()
            pltpu.sync_copy(bufs[b], out_hbm.at[pl.ds(base + gg * _G, _G)])
        return 0

    lax.fori_loop(0, _NG // 2, g_body, 0)


@jax.jit
def _lookup(tid, role_flat, tbl_flat):
    mesh = plsc.VectorSubcoreMesh(core_axis_name="c", subcore_axis_name="s")
    kfn = functools.partial(
        pl.kernel,
        mesh=mesh,
        compiler_params=pltpu.CompilerParams(needs_layout_passes=False),
        out_type=jax.ShapeDtypeStruct((_B * _N, _ROLE_DIM), jnp.float32),
        scratch_types=[
            pltpu.VMEM((_RPW,), jnp.int32),
            pltpu.VMEM((_CHUNK,), jnp.int32),
            pltpu.VMEM((_NG, _G), jnp.int32),
            pltpu.VMEM((_G, _ROLE_DIM), jnp.float32),
            pltpu.VMEM((_G, _ROLE_DIM), jnp.float32),
            pltpu.SemaphoreType.DMA,
            pltpu.SemaphoreType.DMA,
        ],
    )(_sc_body)
    return kfn(tid, role_flat, tbl_flat)


def kernel(template_id_int, atom_role, tables):
    tid = template_id_int.astype(jnp.int32)
    role_flat = atom_role.astype(jnp.int32).reshape(_B * _N)
    tbl_flat = tables.reshape(_NUM_TEMPLATES * _NUM_ROLES, _ROLE_DIM)
    out = _lookup(tid, role_flat, tbl_flat)
    return out.reshape(_B, _N, _ROLE_DIM)


# speedup vs baseline: 3.6161x; 1.7680x over previous
import functools

import jax
import jax.numpy as jnp
from jax import lax
from jax.experimental import pallas as pl
from jax.experimental.pallas import tpu as pltpu
from jax.experimental.pallas import tpu_sc as plsc

_NUM_TEMPLATES = 1000
_NUM_ROLES = 64
_ROLE_DIM = 128
_B = 4096
_N = 50

_NW = 32
_RPW = _B // _NW
_CHUNK = _RPW * _N
_SPG = 2
_G = _SPG * _N
_NG = _RPW // _SPG
_NBUF = 6
_LEAD = 3
_COLS = (0, 16, 32, 48, 64, 80, _G - 16)
_MAGIC, _SHIFT = 41944, 21


def _sc_body(tid_hbm, role_hbm, tbl_hbm, out_hbm,
             tid_v, role_v, idx_v, bufs, gsems, ssems):
    cid = lax.axis_index("c")
    sid = lax.axis_index("s")
    wid = cid * 16 + sid
    base = wid * _CHUNK
    sbase = wid * _RPW

    pltpu.sync_copy(tid_hbm.at[pl.ds(sbase, _RPW)], tid_v)
    pltpu.sync_copy(role_hbm.at[pl.ds(base, _CHUNK)], role_v)

    lanes = lax.iota(jnp.int32, 16)

    def compute_row(g):
        for c in _COLS:
            p0 = g * _G + c
            pv = p0 + lanes
            iv = lax.shift_right_logical(pv * _MAGIC, _SHIFT)
            tv = plsc.load_gather(tid_v, [iv])
            rv = role_v[pl.ds(p0, 16)]
            idx_v[g, pl.ds(c, 16)] = tv * _NUM_ROLES + rv

    def fire_gather(g, b):
        pltpu.async_copy(tbl_hbm.at[idx_v.at[g]], bufs[b], gsems[b])

    def wait_gather(g, b):
        pltpu.make_async_copy(
            tbl_hbm.at[idx_v.at[g]], bufs[b], gsems[b]).wait()

    def fire_scatter(g, b):
        s0 = sbase + g * _SPG
        pltpu.async_copy(bufs[b].at[pl.ds(0, _N)], out_hbm.at[s0], ssems[b])
        pltpu.async_copy(
            bufs[b].at[pl.ds(_N, _N)], out_hbm.at[s0 + 1], ssems[b])

    def wait_scatter(b):
        for _ in range(2):
            pltpu.make_async_copy(
                bufs[b].at[pl.ds(0, _N)], out_hbm.at[sbase], ssems[b]).wait()

    def idx_body(g, _):
        compute_row(g)
        return 0

    lax.fori_loop(0, _NG, idx_body, 0)

    for g in range(_LEAD):
        fire_gather(g, g)

    def g_body(i, _):
        for b in range(_NBUF):
            g = i * _NBUF + b
            bf = (b + _LEAD) % _NBUF
            nf = g + _LEAD

            @pl.when(g < _NG)
            def _():
                @pl.when(jnp.logical_and(nf < _NG, nf >= _NBUF))
                def _():
                    wait_scatter(bf)

                @pl.when(nf < _NG)
                def _():
                    fire_gather(nf, bf)

                wait_gather(g, b)
                fire_scatter(g, b)
        return 0

    lax.fori_loop(0, (_NG + _NBUF - 1) // _NBUF, g_body, 0)

    for b in range(_NBUF):
        wait_scatter(b)


@jax.jit
def _lookup(tid, role_flat, tbl_flat):
    mesh = plsc.VectorSubcoreMesh(core_axis_name="c", subcore_axis_name="s")
    kfn = functools.partial(
        pl.kernel,
        mesh=mesh,
        compiler_params=pltpu.CompilerParams(needs_layout_passes=False),
        out_type=jax.ShapeDtypeStruct((_B, _N, _ROLE_DIM), jnp.float32),
        scratch_types=[
            pltpu.VMEM((_RPW,), jnp.int32),
            pltpu.VMEM((_CHUNK,), jnp.int32),
            pltpu.VMEM((_NG, _G), jnp.int32),
            [pltpu.VMEM((_G, _ROLE_DIM), jnp.float32) for _ in range(_NBUF)],
            [pltpu.SemaphoreType.DMA for _ in range(_NBUF)],
            [pltpu.SemaphoreType.DMA for _ in range(_NBUF)],
        ],
    )(_sc_body)
    return kfn(tid, role_flat, tbl_flat)


def kernel(template_id_int, atom_role, tables):
    tid = template_id_int.astype(jnp.int32)
    role_flat = atom_role.astype(jnp.int32).reshape(_B * _N)
    tbl_flat = tables.reshape(_NUM_TEMPLATES * _NUM_ROLES, _ROLE_DIM)
    return _lookup(tid, role_flat, tbl_flat)


# speedup vs baseline: 6.3259x; 1.7494x over previous
import functools

import jax
import jax.numpy as jnp
from jax import lax
from jax.experimental import pallas as pl
from jax.experimental.pallas import tpu as pltpu
from jax.experimental.pallas import tpu_sc as plsc

_NUM_TEMPLATES = 1000
_NUM_ROLES = 64
_ROLE_DIM = 128
_B = 4096
_N = 50

_NW = 32
_CHUNK = _B * _N // _NW
_G = 128
_NG = _CHUNK // _G
_NBUF = 6
_LEAD = 3


def _sc_body(tid_hbm, role_hbm, tbl_hbm, out_hbm,
             tid_v, role_v, idx_v, bufs, gsems, ssems):
    cid = lax.axis_index("c")
    sid = lax.axis_index("s")
    wid = cid * 16 + sid
    base = wid * _CHUNK

    pltpu.sync_copy(tid_hbm, tid_v)
    pltpu.sync_copy(role_hbm.at[pl.ds(base, _CHUNK)], role_v)

    def idx_body(k, _):
        p0 = k * 16
        i0 = (base + p0) & (_B - 1)
        tv = tid_v[pl.ds(i0, 16)]
        rv = role_v[pl.ds(p0, 16)]
        idx_v[k // 8, pl.ds((k % 8) * 16, 16)] = tv * _NUM_ROLES + rv
        return 0

    lax.fori_loop(0, _CHUNK // 16, idx_body, 0)

    def fire_gather(g, b):
        pltpu.async_copy(tbl_hbm.at[idx_v.at[g]], bufs[b], gsems[b])

    def wait_gather(g, b):
        pltpu.make_async_copy(
            tbl_hbm.at[idx_v.at[g]], bufs[b], gsems[b]).wait()

    def fire_scatter(g, b):
        pltpu.async_copy(
            bufs[b], out_hbm.at[pl.ds(base + g * _G, _G)], ssems[b])

    def wait_scatter(b):
        pltpu.make_async_copy(
            bufs[b], out_hbm.at[pl.ds(base, _G)], ssems[b]).wait()

    for g in range(_LEAD):
        fire_gather(g, g)

    def g_body(i, _):
        for b in range(_NBUF):
            g = i * _NBUF + b
            bf = (b + _LEAD) % _NBUF
            nf = g + _LEAD

            @pl.when(g < _NG)
            def _():
                @pl.when(jnp.logical_and(nf < _NG, nf >= _NBUF))
                def _():
                    wait_scatter(bf)

                @pl.when(nf < _NG)
                def _():
                    fire_gather(nf, bf)

                wait_gather(g, b)
                fire_scatter(g, b)
        return 0

    lax.fori_loop(0, (_NG + _NBUF - 1) // _NBUF, g_body, 0)

    for b in range(_NBUF):
        wait_scatter(b)


@jax.jit
def _lookup(tid, role_t_flat, tbl_flat):
    mesh = plsc.VectorSubcoreMesh(core_axis_name="c", subcore_axis_name="s")
    kfn = functools.partial(
        pl.kernel,
        mesh=mesh,
        compiler_params=pltpu.CompilerParams(needs_layout_passes=False),
        out_type=jax.ShapeDtypeStruct((_N * _B, _ROLE_DIM), jnp.float32),
        scratch_types=[
            pltpu.VMEM((_B,), jnp.int32),
            pltpu.VMEM((_CHUNK,), jnp.int32),
            pltpu.VMEM((_NG, _G), jnp.int32),
            [pltpu.VMEM((_G, _ROLE_DIM), jnp.float32) for _ in range(_NBUF)],
            [pltpu.SemaphoreType.DMA for _ in range(_NBUF)],
            [pltpu.SemaphoreType.DMA for _ in range(_NBUF)],
        ],
    )(_sc_body)
    return kfn(tid, role_t_flat, tbl_flat)


def kernel(template_id_int, atom_role, tables):
    tid = template_id_int.astype(jnp.int32)
    role_t_flat = atom_role.astype(jnp.int32).T.reshape(_N * _B)
    tbl_flat = tables.reshape(_NUM_TEMPLATES * _NUM_ROLES, _ROLE_DIM)
    out = _lookup(tid, role_t_flat, tbl_flat)
    return out.reshape(_N, _B, _ROLE_DIM).transpose(1, 0, 2)


# speedup vs baseline: 6.3749x; 1.0077x over previous
import functools

import jax
import jax.numpy as jnp
from jax import lax
from jax.experimental import pallas as pl
from jax.experimental.pallas import tpu as pltpu
from jax.experimental.pallas import tpu_sc as plsc

_NUM_TEMPLATES = 1000
_NUM_ROLES = 64
_ROLE_DIM = 128
_B = 4096
_N = 50

_NW = 32
_CHUNK = _B * _N // _NW
_G = 128
_NG = _CHUNK // _G
_NBUF = 6
_LEAD = 4


def _sc_body(tid_hbm, role_hbm, tbl_hbm, out_hbm,
             tid_v, role_v, idx_v, bufs, gsems, ssems):
    cid = lax.axis_index("c")
    sid = lax.axis_index("s")
    wid = cid * 16 + sid
    base = wid * _CHUNK

    pltpu.sync_copy(tid_hbm, tid_v)
    pltpu.sync_copy(role_hbm.at[pl.ds(base, _CHUNK)], role_v)

    def idx_body(k, _):
        p0 = k * 16
        i0 = (base + p0) & (_B - 1)
        tv = tid_v[pl.ds(i0, 16)]
        rv = role_v[pl.ds(p0, 16)]
        idx_v[k // 8, pl.ds((k % 8) * 16, 16)] = tv * _NUM_ROLES + rv
        return 0

    lax.fori_loop(0, _CHUNK // 16, idx_body, 0)

    def fire_gather(g, b):
        pltpu.async_copy(tbl_hbm.at[idx_v.at[g]], bufs[b], gsems[b])

    def wait_gather(g, b):
        pltpu.make_async_copy(
            tbl_hbm.at[idx_v.at[g]], bufs[b], gsems[b]).wait()

    def fire_scatter(g, b):
        pltpu.async_copy(
            bufs[b], out_hbm.at[pl.ds(base + g * _G, _G)], ssems[b])

    def wait_scatter(b):
        pltpu.make_async_copy(
            bufs[b], out_hbm.at[pl.ds(base, _G)], ssems[b]).wait()

    for g in range(_LEAD):
        fire_gather(g, g)

    def g_body(i, _):
        for b in range(_NBUF):
            g = i * _NBUF + b
            bf = (b + _LEAD) % _NBUF
            nf = g + _LEAD

            @pl.when(g < _NG)
            def _():
                @pl.when(jnp.logical_and(nf < _NG, nf >= _NBUF))
                def _():
                    wait_scatter(bf)

                @pl.when(nf < _NG)
                def _():
                    fire_gather(nf, bf)

                wait_gather(g, b)
                fire_scatter(g, b)
        return 0

    lax.fori_loop(0, (_NG + _NBUF - 1) // _NBUF, g_body, 0)

    for b in range(_NBUF):
        wait_scatter(b)


@jax.jit
def _lookup(tid, role_t_flat, tbl_flat):
    mesh = plsc.VectorSubcoreMesh(core_axis_name="c", subcore_axis_name="s")
    kfn = functools.partial(
        pl.kernel,
        mesh=mesh,
        compiler_params=pltpu.CompilerParams(needs_layout_passes=False),
        out_type=jax.ShapeDtypeStruct((_N * _B, _ROLE_DIM), jnp.float32),
        scratch_types=[
            pltpu.VMEM((_B,), jnp.int32),
            pltpu.VMEM((_CHUNK,), jnp.int32),
            pltpu.VMEM((_NG, _G), jnp.int32),
            [pltpu.VMEM((_G, _ROLE_DIM), jnp.float32) for _ in range(_NBUF)],
            [pltpu.SemaphoreType.DMA for _ in range(_NBUF)],
            [pltpu.SemaphoreType.DMA for _ in range(_NBUF)],
        ],
    )(_sc_body)
    return kfn(tid, role_t_flat, tbl_flat)


def kernel(template_id_int, atom_role, tables):
    tid = template_id_int.astype(jnp.int32)
    role_t_flat = atom_role.astype(jnp.int32).T.reshape(_N * _B)
    tbl_flat = tables.reshape(_NUM_TEMPLATES * _NUM_ROLES, _ROLE_DIM)
    out = _lookup(tid, role_t_flat, tbl_flat)
    return out.reshape(_N, _B, _ROLE_DIM).transpose(1, 0, 2)


# speedup vs baseline: 6.3883x; 1.0021x over previous
import functools

import jax
import jax.numpy as jnp
from jax import lax
from jax.experimental import pallas as pl
from jax.experimental.pallas import tpu as pltpu
from jax.experimental.pallas import tpu_sc as plsc

_NUM_TEMPLATES = 1000
_NUM_ROLES = 64
_ROLE_DIM = 128
_B = 4096
_N = 50

_NW = 32
_CHUNK = _B * _N // _NW
_G = 80
_NG = _CHUNK // _G
_NBUF = 10
_LEAD = 6


def _sc_body(tid_hbm, role_hbm, tbl_hbm, out_hbm,
             tid_v, role_v, idx_v, bufs, gsems, ssems):
    cid = lax.axis_index("c")
    sid = lax.axis_index("s")
    wid = cid * 16 + sid
    base = wid * _CHUNK

    pltpu.sync_copy(tid_hbm, tid_v)
    pltpu.sync_copy(role_hbm.at[pl.ds(base, _CHUNK)], role_v)

    def idx_body(k, _):
        p0 = k * 16
        i0 = (base + p0) & (_B - 1)
        tv = tid_v[pl.ds(i0, 16)]
        rv = role_v[pl.ds(p0, 16)]
        idx_v[k // (_G // 16), pl.ds((k % (_G // 16)) * 16, 16)] = tv * _NUM_ROLES + rv
        return 0

    lax.fori_loop(0, _CHUNK // 16, idx_body, 0)

    def fire_gather(g, b):
        pltpu.async_copy(tbl_hbm.at[idx_v.at[g]], bufs[b], gsems[b])

    def wait_gather(g, b):
        pltpu.make_async_copy(
            tbl_hbm.at[idx_v.at[g]], bufs[b], gsems[b]).wait()

    def fire_scatter(g, b):
        pltpu.async_copy(
            bufs[b], out_hbm.at[pl.ds(base + g * _G, _G)], ssems[b])

    def wait_scatter(b):
        pltpu.make_async_copy(
            bufs[b], out_hbm.at[pl.ds(base, _G)], ssems[b]).wait()

    for g in range(_LEAD):
        fire_gather(g, g)

    def g_body(i, _):
        for b in range(_NBUF):
            g = i * _NBUF + b
            bf = (b + _LEAD) % _NBUF
            nf = g + _LEAD

            @pl.when(g < _NG)
            def _():
                @pl.when(jnp.logical_and(nf < _NG, nf >= _NBUF))
                def _():
                    wait_scatter(bf)

                @pl.when(nf < _NG)
                def _():
                    fire_gather(nf, bf)

                wait_gather(g, b)
                fire_scatter(g, b)
        return 0

    lax.fori_loop(0, (_NG + _NBUF - 1) // _NBUF, g_body, 0)

    for b in range(_NBUF):
        wait_scatter(b)


@jax.jit
def _lookup(tid, role_t_flat, tbl_flat):
    mesh = plsc.VectorSubcoreMesh(core_axis_name="c", subcore_axis_name="s")
    kfn = functools.partial(
        pl.kernel,
        mesh=mesh,
        compiler_params=pltpu.CompilerParams(needs_layout_passes=False),
        out_type=jax.ShapeDtypeStruct((_N * _B, _ROLE_DIM), jnp.float32),
        scratch_types=[
            pltpu.VMEM((_B,), jnp.int32),
            pltpu.VMEM((_CHUNK,), jnp.int32),
            pltpu.VMEM((_NG, _G), jnp.int32),
            [pltpu.VMEM((_G, _ROLE_DIM), jnp.float32) for _ in range(_NBUF)],
            [pltpu.SemaphoreType.DMA for _ in range(_NBUF)],
            [pltpu.SemaphoreType.DMA for _ in range(_NBUF)],
        ],
    )(_sc_body)
    return kfn(tid, role_t_flat, tbl_flat)


def kernel(template_id_int, atom_role, tables):
    tid = template_id_int.astype(jnp.int32)
    role_t_flat = atom_role.astype(jnp.int32).T.reshape(_N * _B)
    tbl_flat = tables.reshape(_NUM_TEMPLATES * _NUM_ROLES, _ROLE_DIM)
    out = _lookup(tid, role_t_flat, tbl_flat)
    return out.reshape(_N, _B, _ROLE_DIM).transpose(1, 0, 2)
